# baseline (speedup 1.0000x reference)
import jax
import jax.numpy as jnp
from jax.experimental import pallas as pl

_BT = 1024


def _gate_body(x_ref, wt_ref, bias_ref, idx_ref, score_ref):
    x = x_ref[...]
    logits = jnp.dot(x, wt_ref[...], preferred_element_type=jnp.float32)
    logits = logits + bias_ref[...]
    m = jnp.max(logits, axis=1, keepdims=True)
    e = jnp.exp(logits - m)
    s = jnp.sum(e, axis=1, keepdims=True)
    sm = e / s
    v = 1.0 / s
    lane = jax.lax.broadcasted_iota(jnp.int32, sm.shape, 1).astype(jnp.float32)
    idxf = jnp.min(jnp.where(sm >= v, lane, float(sm.shape[1])),
                   axis=1, keepdims=True)
    idx_ref[...] = idxf.astype(jnp.int32)
    score_ref[...] = v


def kernel(inp, W, b):
    T, D = inp.shape
    E = W.shape[0]
    wt = W.T
    bias = b.reshape(1, E)
    grid = (T // _BT,)
    idx, score = pl.pallas_call(
        _gate_body,
        grid=grid,
        in_specs=[
            pl.BlockSpec((_BT, D), lambda i: (i, 0)),
            pl.BlockSpec((D, E), lambda i: (0, 0)),
            pl.BlockSpec((1, E), lambda i: (0, 0)),
        ],
        out_specs=[
            pl.BlockSpec((_BT, 1), lambda i: (i, 0)),
            pl.BlockSpec((_BT, 1), lambda i: (i, 0)),
        ],
        out_shape=[
            jax.ShapeDtypeStruct((T, 1), jnp.int32),
            jax.ShapeDtypeStruct((T, 1), jnp.float32),
        ],
    )(inp, wt, bias)
    return (idx.astype(jnp.int64), score)


# speedup vs baseline: 1.0306x; 1.0306x over previous
import jax
import jax.numpy as jnp
from jax.experimental import pallas as pl

_BT = 4096


def _gate_body(x_ref, wt_ref, bias_ref, idx_ref, score_ref):
    x = x_ref[...]
    logits = jnp.dot(x, wt_ref[...], preferred_element_type=jnp.float32)
    logits = logits + bias_ref[...]
    m = jnp.max(logits, axis=1, keepdims=True)
    e = jnp.exp(logits - m)
    s = jnp.sum(e, axis=1, keepdims=True)
    sm = e / s
    v = 1.0 / s
    lane = jax.lax.broadcasted_iota(jnp.int32, sm.shape, 1).astype(jnp.float32)
    idxf = jnp.min(jnp.where(sm >= v, lane, float(sm.shape[1])),
                   axis=1, keepdims=True)
    idx_ref[...] = idxf.astype(jnp.int32)
    score_ref[...] = v


def kernel(inp, W, b):
    T, D = inp.shape
    E = W.shape[0]
    wt = W.T
    bias = b.reshape(1, E)
    grid = (T // _BT,)
    idx, score = pl.pallas_call(
        _gate_body,
        grid=grid,
        in_specs=[
            pl.BlockSpec((_BT, D), lambda i: (i, 0)),
            pl.BlockSpec((D, E), lambda i: (0, 0)),
            pl.BlockSpec((1, E), lambda i: (0, 0)),
        ],
        out_specs=[
            pl.BlockSpec((_BT, 1), lambda i: (i, 0)),
            pl.BlockSpec((_BT, 1), lambda i: (i, 0)),
        ],
        out_shape=[
            jax.ShapeDtypeStruct((T, 1), jnp.int32),
            jax.ShapeDtypeStruct((T, 1), jnp.float32),
        ],
    )(inp, wt, bias)
    return (idx.astype(jnp.int64), score)


# speedup vs baseline: 1.0623x; 1.0308x over previous
import jax
import jax.numpy as jnp
from jax.experimental import pallas as pl

_BT = 2048


def _gate_body(x_ref, wt_ref, bias_ref, idx_ref, score_ref):
    x = x_ref[...]
    logits = jnp.dot(x, wt_ref[...], preferred_element_type=jnp.float32)
    logits = logits + bias_ref[...]
    m = jnp.max(logits, axis=1, keepdims=True)
    e = jnp.exp(logits - m)
    s = jnp.sum(e, axis=1, keepdims=True)
    sm = e / s
    v = 1.0 / s
    lane = jax.lax.broadcasted_iota(jnp.int32, sm.shape, 1).astype(jnp.float32)
    idxf = jnp.min(jnp.where(sm >= v, lane, float(sm.shape[1])),
                   axis=1, keepdims=True)
    idx_ref[...] = idxf.astype(jnp.int32)
    score_ref[...] = v


def kernel(inp, W, b):
    T, D = inp.shape
    E = W.shape[0]
    wt = W.T
    bias = b.reshape(1, E)
    grid = (T // _BT,)
    idx, score = pl.pallas_call(
        _gate_body,
        grid=grid,
        in_specs=[
            pl.BlockSpec((_BT, D), lambda i: (i, 0)),
            pl.BlockSpec((D, E), lambda i: (0, 0)),
            pl.BlockSpec((1, E), lambda i: (0, 0)),
        ],
        out_specs=[
            pl.BlockSpec((_BT, 1), lambda i: (i, 0)),
            pl.BlockSpec((_BT, 1), lambda i: (i, 0)),
        ],
        out_shape=[
            jax.ShapeDtypeStruct((T, 1), jnp.int32),
            jax.ShapeDtypeStruct((T, 1), jnp.float32),
        ],
    )(inp, wt, bias)
    return (idx.astype(jnp.int64), score)
